# Initial kernel scaffold; baseline (speedup 1.0000x reference)
import functools

import jax
import jax.numpy as jnp
import numpy as np
from jax import lax
from jax.experimental import pallas as pl
from jax.experimental.pallas import tpu as pltpu
from jax.experimental.pallas import tpu_sc as plsc

N_NODES = 10000
N_EDGES = 320000
D_FEAT = 128
NPAD = 10240
NC, NS = 2, 16
NW = NC * NS
E_TILE = 10240
EPAD = NW * E_TILE
ROWS = 16
LANES = 128
N_CHUNK = E_TILE // (ROWS * LANES)
N_SLICE = NPAD // NS

CUTOFF_DISTANCE = 4.0
CUTOFF_ONSET = 3.8
_D = 0.5 * (CUTOFF_DISTANCE - CUTOFF_ONSET)
_R = CUTOFF_DISTANCE - _D

_NODE_BLK = 1000
_EDGE_BLK = EPAD // 128 // 10


def _tc_pre_body(nf_ref, wsT_ref, bs_ref, wdT_ref, bo_ref, bl_ref,
                 es_ref, ed_ref, a_ref, b_ref):
    x = nf_ref[...]
    es_ref[...] = jnp.exp(
        jnp.dot(x, wsT_ref[...], preferred_element_type=jnp.float32)
        + bs_ref[...])
    ed_ref[...] = jnp.exp(
        jnp.dot(x, wdT_ref[...], preferred_element_type=jnp.float32))
    r = bl_ref[...]
    c = jnp.where(r < _R - _D, 1.0,
                  0.5 - 0.5 * jnp.sin(np.pi * (r - _R) / (2 * _D)))
    c = jnp.where(r > _R + _D, 0.0, c)
    a_ref[...] = c
    b_ref[...] = c * bo_ref[...]


_tc_pre = pl.pallas_call(
    _tc_pre_body,
    grid=(10,),
    in_specs=[
        pl.BlockSpec((_NODE_BLK, D_FEAT), lambda i: (i, 0)),
        pl.BlockSpec((D_FEAT, 4), lambda i: (0, 0)),
        pl.BlockSpec((1, 4), lambda i: (0, 0)),
        pl.BlockSpec((D_FEAT, 4), lambda i: (0, 0)),
        pl.BlockSpec((_EDGE_BLK, LANES), lambda i: (i, 0)),
        pl.BlockSpec((_EDGE_BLK, LANES), lambda i: (i, 0)),
    ],
    out_specs=[
        pl.BlockSpec((_NODE_BLK, 4), lambda i: (i, 0)),
        pl.BlockSpec((_NODE_BLK, 4), lambda i: (i, 0)),
        pl.BlockSpec((_EDGE_BLK, LANES), lambda i: (i, 0)),
        pl.BlockSpec((_EDGE_BLK, LANES), lambda i: (i, 0)),
    ],
    out_shape=[
        jax.ShapeDtypeStruct((N_NODES, 4), jnp.float32),
        jax.ShapeDtypeStruct((N_NODES, 4), jnp.float32),
        jax.ShapeDtypeStruct((EPAD // 128, LANES), jnp.float32),
        jax.ShapeDtypeStruct((EPAD // 128, LANES), jnp.float32),
    ],
)


@functools.partial(
    pl.kernel,
    mesh=plsc.VectorSubcoreMesh(core_axis_name="c", subcore_axis_name="s"),
    out_type=jax.ShapeDtypeStruct((NC, NPAD), jnp.float32),
    scratch_types=[
        pltpu.VMEM((NPAD * 4,), jnp.float32),
        pltpu.VMEM((NPAD * 4,), jnp.float32),
        pltpu.VMEM((ROWS, LANES), jnp.int32),
        pltpu.VMEM((ROWS, LANES), jnp.int32),
        pltpu.VMEM((ROWS, LANES), jnp.float32),
        pltpu.VMEM((ROWS, LANES), jnp.float32),
        pltpu.VMEM((ROWS, LANES), jnp.float32),
        pltpu.VMEM((ROWS, LANES), jnp.float32),
        pltpu.VMEM((N_SLICE,), jnp.float32),
        pltpu.VMEM_SHARED((NPAD,), jnp.float32),
    ],
)
def _sc_edges(es_hbm, ed_hbm, src_hbm, dst_hbm, bl_hbm, a_hbm, b_hbm,
              out_hbm, es_v, ed_v, src_v, dst_v, bl_v, a_v, b_v, vp_v,
              zero_v, acc_sp):
    cid = lax.axis_index("c")
    sid = lax.axis_index("s")
    wid = sid * NC + cid

    def _zero(i, carry):
        zero_v[pl.ds(i * 16, 16)] = jnp.zeros((16,), jnp.float32)
        return carry
    lax.fori_loop(0, N_SLICE // 16, _zero, 0)
    pltpu.sync_copy(zero_v, acc_sp.at[pl.ds(sid * N_SLICE, N_SLICE)])

    pltpu.sync_copy(es_hbm, es_v)
    pltpu.sync_copy(ed_hbm, ed_v)
    plsc.subcore_barrier()

    def _chunk(ch, carry):
        pltpu.sync_copy(src_hbm.at[wid, ch], src_v)
        pltpu.sync_copy(dst_hbm.at[wid, ch], dst_v)
        pltpu.sync_copy(bl_hbm.at[wid, ch], bl_v)
        pltpu.sync_copy(a_hbm.at[wid, ch], a_v)
        pltpu.sync_copy(b_hbm.at[wid, ch], b_v)
        for r in range(ROWS):
            for g in range(LANES // 16):
                sl = pl.ds(g * 16, 16)
                s4 = src_v[r, sl] * 4
                d4 = dst_v[r, sl] * 4
                es0 = plsc.load_gather(es_v, [s4])
                es1 = plsc.load_gather(es_v, [s4 + 1])
                es2 = plsc.load_gather(es_v, [s4 + 2])
                es3 = plsc.load_gather(es_v, [s4 + 3])
                ed0 = plsc.load_gather(ed_v, [d4])
                ed1 = plsc.load_gather(ed_v, [d4 + 1])
                ed2 = plsc.load_gather(ed_v, [d4 + 2])
                ed3 = plsc.load_gather(ed_v, [d4 + 3])
                rr = bl_v[r, sl]
                rep = a_v[r, sl] * es0 * ed0 * jnp.exp(-es1 * ed1 * rr)
                att = b_v[r, sl] * es2 * ed2 * jnp.exp(-es3 * ed3 * rr)
                vp_v[r, sl] = rep - att
            pltpu.sync_copy(vp_v.at[r], acc_sp.at[dst_v.at[r]], add=True)
        return carry
    lax.fori_loop(0, N_CHUNK, _chunk, 0)

    plsc.subcore_barrier()
    pltpu.sync_copy(acc_sp.at[pl.ds(sid * N_SLICE, N_SLICE)],
                    out_hbm.at[cid, pl.ds(sid * N_SLICE, N_SLICE)])


def _tc_sum_body(x_ref, o_ref):
    o_ref[...] = x_ref[0, :] + x_ref[1, :]


_tc_sum = pl.pallas_call(
    _tc_sum_body,
    out_shape=jax.ShapeDtypeStruct((NPAD,), jnp.float32),
)


def kernel(node_features, bond_order, bondlength, edge_index, W_src, b_src, W_dst):
    e_extra = EPAD - N_EDGES
    bo_p = jnp.concatenate(
        [bond_order, jnp.zeros((e_extra,), jnp.float32)])
    bl_p = jnp.concatenate(
        [bondlength, jnp.full((e_extra,), 100.0, jnp.float32)])
    es, ed, a2, b2 = _tc_pre(
        node_features, W_src.T, b_src.reshape(1, 4), W_dst.T,
        bo_p.reshape(EPAD // 128, LANES), bl_p.reshape(EPAD // 128, LANES))

    es_f = jnp.pad(es, ((0, NPAD - N_NODES), (0, 0))).reshape(-1)
    ed_f = jnp.pad(ed, ((0, NPAD - N_NODES), (0, 0))).reshape(-1)
    zpad = jnp.zeros((e_extra,), jnp.int32)
    shp = (NW, N_CHUNK, ROWS, LANES)
    src_r = jnp.concatenate([edge_index[0], zpad]).reshape(shp)
    dst_r = jnp.concatenate([edge_index[1], zpad]).reshape(shp)
    bl_r = bl_p.reshape(shp)
    a_r = a2.reshape(shp)
    b_r = b2.reshape(shp)

    out2 = _sc_edges(es_f, ed_f, src_r, dst_r, bl_r, a_r, b_r)
    return _tc_sum(out2)[:N_NODES]


# speedup vs baseline: 16.3802x; 16.3802x over previous
import functools

import jax
import jax.numpy as jnp
import numpy as np
from jax import lax
from jax.experimental import pallas as pl
from jax.experimental.pallas import tpu as pltpu
from jax.experimental.pallas import tpu_sc as plsc

N_NODES = 10000
N_EDGES = 320000
D_FEAT = 128
NPAD = 10240
NC, NS = 2, 16
NW = NC * NS
E_TILE = 10240
EPAD = NW * E_TILE
ROWS = 16
LANES = 128
N_CHUNK = E_TILE // (ROWS * LANES)
N_SLICE = NPAD // NS

CUTOFF_DISTANCE = 4.0
CUTOFF_ONSET = 3.8
_D = 0.5 * (CUTOFF_DISTANCE - CUTOFF_ONSET)
_R = CUTOFF_DISTANCE - _D

_NODE_BLK = 1000
_EDGE_BLK = EPAD // 128 // 10


def _tc_pre_body(nf_ref, wsT_ref, bs_ref, wdT_ref, bo_ref, bl_ref,
                 es_ref, ed_ref, a_ref, b_ref):
    x = nf_ref[...]
    es_ref[...] = jnp.exp(
        jnp.dot(x, wsT_ref[...], preferred_element_type=jnp.float32)
        + bs_ref[...])
    ed_ref[...] = jnp.exp(
        jnp.dot(x, wdT_ref[...], preferred_element_type=jnp.float32))
    r = bl_ref[...]
    c = jnp.where(r < _R - _D, 1.0,
                  0.5 - 0.5 * jnp.sin(np.pi * (r - _R) / (2 * _D)))
    c = jnp.where(r > _R + _D, 0.0, c)
    a_ref[...] = c
    b_ref[...] = c * bo_ref[...]


_tc_pre = pl.pallas_call(
    _tc_pre_body,
    grid=(10,),
    in_specs=[
        pl.BlockSpec((_NODE_BLK, D_FEAT), lambda i: (i, 0)),
        pl.BlockSpec((D_FEAT, 4), lambda i: (0, 0)),
        pl.BlockSpec((1, 4), lambda i: (0, 0)),
        pl.BlockSpec((D_FEAT, 4), lambda i: (0, 0)),
        pl.BlockSpec((_EDGE_BLK, LANES), lambda i: (i, 0)),
        pl.BlockSpec((_EDGE_BLK, LANES), lambda i: (i, 0)),
    ],
    out_specs=[
        pl.BlockSpec((_NODE_BLK, 4), lambda i: (i, 0)),
        pl.BlockSpec((_NODE_BLK, 4), lambda i: (i, 0)),
        pl.BlockSpec((_EDGE_BLK, LANES), lambda i: (i, 0)),
        pl.BlockSpec((_EDGE_BLK, LANES), lambda i: (i, 0)),
    ],
    out_shape=[
        jax.ShapeDtypeStruct((N_NODES, 4), jnp.float32),
        jax.ShapeDtypeStruct((N_NODES, 4), jnp.float32),
        jax.ShapeDtypeStruct((EPAD // 128, LANES), jnp.float32),
        jax.ShapeDtypeStruct((EPAD // 128, LANES), jnp.float32),
    ],
)


@functools.partial(
    pl.kernel,
    mesh=plsc.VectorSubcoreMesh(core_axis_name="c", subcore_axis_name="s"),
    compiler_params=pltpu.CompilerParams(needs_layout_passes=False),
    out_type=jax.ShapeDtypeStruct((NC, NPAD), jnp.float32),
    scratch_types=[
        pltpu.VMEM((NPAD * 4,), jnp.float32),
        pltpu.VMEM((NPAD * 4,), jnp.float32),
        pltpu.VMEM((ROWS, LANES), jnp.int32),
        pltpu.VMEM((ROWS, LANES), jnp.int32),
        pltpu.VMEM((ROWS, LANES), jnp.float32),
        pltpu.VMEM((ROWS, LANES), jnp.float32),
        pltpu.VMEM((ROWS, LANES), jnp.float32),
        pltpu.VMEM((ROWS, LANES), jnp.float32),
        pltpu.VMEM((N_SLICE,), jnp.float32),
        pltpu.VMEM_SHARED((NPAD,), jnp.float32),
    ],
)
def _sc_edges(es_hbm, ed_hbm, src_hbm, dst_hbm, bl_hbm, a_hbm, b_hbm,
              out_hbm, es_v, ed_v, src_v, dst_v, bl_v, a_v, b_v, vp_v,
              zero_v, acc_sp):
    cid = lax.axis_index("c")
    sid = lax.axis_index("s")
    wid = sid * NC + cid

    def _zero(i, carry):
        zero_v[pl.ds(i * 16, 16)] = jnp.zeros((16,), jnp.float32)
        return carry
    lax.fori_loop(0, N_SLICE // 16, _zero, 0)
    pltpu.sync_copy(zero_v, acc_sp.at[pl.ds(sid * N_SLICE, N_SLICE)])

    pltpu.sync_copy(es_hbm, es_v)
    pltpu.sync_copy(ed_hbm, ed_v)
    plsc.subcore_barrier()

    def _chunk(ch, carry):
        pltpu.sync_copy(src_hbm.at[wid, ch], src_v)
        pltpu.sync_copy(dst_hbm.at[wid, ch], dst_v)
        pltpu.sync_copy(bl_hbm.at[wid, ch], bl_v)
        pltpu.sync_copy(a_hbm.at[wid, ch], a_v)
        pltpu.sync_copy(b_hbm.at[wid, ch], b_v)
        for r in range(ROWS):
            for g in range(LANES // 16):
                sl = pl.ds(g * 16, 16)
                s4 = src_v[r, sl] * 4
                d4 = dst_v[r, sl] * 4
                es0 = plsc.load_gather(es_v, [s4])
                es1 = plsc.load_gather(es_v, [s4 + 1])
                es2 = plsc.load_gather(es_v, [s4 + 2])
                es3 = plsc.load_gather(es_v, [s4 + 3])
                ed0 = plsc.load_gather(ed_v, [d4])
                ed1 = plsc.load_gather(ed_v, [d4 + 1])
                ed2 = plsc.load_gather(ed_v, [d4 + 2])
                ed3 = plsc.load_gather(ed_v, [d4 + 3])
                rr = bl_v[r, sl]
                rep = a_v[r, sl] * es0 * ed0 * jnp.exp(-es1 * ed1 * rr)
                att = b_v[r, sl] * es2 * ed2 * jnp.exp(-es3 * ed3 * rr)
                vp_v[r, sl] = rep - att
            pltpu.sync_copy(vp_v.at[r], acc_sp.at[dst_v.at[r]], add=True)
        return carry
    lax.fori_loop(0, N_CHUNK, _chunk, 0)

    plsc.subcore_barrier()
    pltpu.sync_copy(acc_sp.at[pl.ds(sid * N_SLICE, N_SLICE)],
                    out_hbm.at[cid, pl.ds(sid * N_SLICE, N_SLICE)])


def _tc_sum_body(x_ref, o_ref):
    o_ref[...] = x_ref[0, :] + x_ref[1, :]


_tc_sum = pl.pallas_call(
    _tc_sum_body,
    out_shape=jax.ShapeDtypeStruct((NPAD,), jnp.float32),
)


def kernel(node_features, bond_order, bondlength, edge_index, W_src, b_src, W_dst):
    e_extra = EPAD - N_EDGES
    bo_p = jnp.concatenate(
        [bond_order, jnp.zeros((e_extra,), jnp.float32)])
    bl_p = jnp.concatenate(
        [bondlength, jnp.full((e_extra,), 100.0, jnp.float32)])
    es, ed, a2, b2 = _tc_pre(
        node_features, W_src.T, b_src.reshape(1, 4), W_dst.T,
        bo_p.reshape(EPAD // 128, LANES), bl_p.reshape(EPAD // 128, LANES))

    es_f = jnp.pad(es, ((0, NPAD - N_NODES), (0, 0))).reshape(-1)
    ed_f = jnp.pad(ed, ((0, NPAD - N_NODES), (0, 0))).reshape(-1)
    zpad = jnp.zeros((e_extra,), jnp.int32)
    shp = (NW, N_CHUNK, ROWS, LANES)
    src_r = jnp.concatenate([edge_index[0], zpad]).reshape(shp)
    dst_r = jnp.concatenate([edge_index[1], zpad]).reshape(shp)
    bl_r = bl_p.reshape(shp)
    a_r = a2.reshape(shp)
    b_r = b2.reshape(shp)

    out2 = _sc_edges(es_f, ed_f, src_r, dst_r, bl_r, a_r, b_r)
    return _tc_sum(out2)[:N_NODES]


# speedup vs baseline: 19.1618x; 1.1698x over previous
import functools

import jax
import jax.numpy as jnp
import numpy as np
from jax import lax
from jax.experimental import pallas as pl
from jax.experimental.pallas import tpu as pltpu
from jax.experimental.pallas import tpu_sc as plsc

N_NODES = 10000
N_EDGES = 320000
D_FEAT = 128
NPAD = 10240
NC, NS = 2, 16
NW = NC * NS
E_TILE = 10240
EPAD = NW * E_TILE
ROWS = 8
LANES = 128
N_CHUNK = E_TILE // (ROWS * LANES)
N_PAIR = N_CHUNK // 2
N_SLICE = NPAD // NS

CUTOFF_DISTANCE = 4.0
CUTOFF_ONSET = 3.8
_D = 0.5 * (CUTOFF_DISTANCE - CUTOFF_ONSET)
_R = CUTOFF_DISTANCE - _D

_NODE_BLK = 1000
_EDGE_BLK = EPAD // 128 // 10


def _tc_pre_body(nf_ref, wsT_ref, bs_ref, wdT_ref, bo_ref, bl_ref,
                 es_ref, ed_ref, a_ref, b_ref):
    x = nf_ref[...]
    es_ref[...] = jnp.exp(
        jnp.dot(x, wsT_ref[...], preferred_element_type=jnp.float32)
        + bs_ref[...])
    ed_ref[...] = jnp.exp(
        jnp.dot(x, wdT_ref[...], preferred_element_type=jnp.float32))
    r = bl_ref[...]
    c = jnp.where(r < _R - _D, 1.0,
                  0.5 - 0.5 * jnp.sin(np.pi * (r - _R) / (2 * _D)))
    c = jnp.where(r > _R + _D, 0.0, c)
    a_ref[...] = c
    b_ref[...] = c * bo_ref[...]


_tc_pre = pl.pallas_call(
    _tc_pre_body,
    grid=(10,),
    in_specs=[
        pl.BlockSpec((_NODE_BLK, D_FEAT), lambda i: (i, 0)),
        pl.BlockSpec((D_FEAT, 4), lambda i: (0, 0)),
        pl.BlockSpec((1, 4), lambda i: (0, 0)),
        pl.BlockSpec((D_FEAT, 4), lambda i: (0, 0)),
        pl.BlockSpec((_EDGE_BLK, LANES), lambda i: (i, 0)),
        pl.BlockSpec((_EDGE_BLK, LANES), lambda i: (i, 0)),
    ],
    out_specs=[
        pl.BlockSpec((_NODE_BLK, 4), lambda i: (i, 0)),
        pl.BlockSpec((_NODE_BLK, 4), lambda i: (i, 0)),
        pl.BlockSpec((_EDGE_BLK, LANES), lambda i: (i, 0)),
        pl.BlockSpec((_EDGE_BLK, LANES), lambda i: (i, 0)),
    ],
    out_shape=[
        jax.ShapeDtypeStruct((N_NODES, 4), jnp.float32),
        jax.ShapeDtypeStruct((N_NODES, 4), jnp.float32),
        jax.ShapeDtypeStruct((EPAD // 128, LANES), jnp.float32),
        jax.ShapeDtypeStruct((EPAD // 128, LANES), jnp.float32),
    ],
)


def _half_chunk_scratch():
    return [
        pltpu.VMEM((ROWS, LANES), jnp.int32),
        pltpu.VMEM((ROWS, LANES), jnp.int32),
        pltpu.VMEM((ROWS, LANES), jnp.float32),
        pltpu.VMEM((ROWS, LANES), jnp.float32),
        pltpu.VMEM((ROWS, LANES), jnp.float32),
        pltpu.VMEM((ROWS, LANES), jnp.float32),
    ]


@functools.partial(
    pl.kernel,
    mesh=plsc.VectorSubcoreMesh(core_axis_name="c", subcore_axis_name="s"),
    compiler_params=pltpu.CompilerParams(needs_layout_passes=False),
    out_type=jax.ShapeDtypeStruct((NC, NPAD), jnp.float32),
    scratch_types=[
        pltpu.VMEM((NPAD * 4,), jnp.float32),
        pltpu.VMEM((NPAD * 4,), jnp.float32),
        *_half_chunk_scratch(),
        *_half_chunk_scratch(),
        pltpu.VMEM((N_SLICE,), jnp.float32),
        pltpu.VMEM_SHARED((NPAD,), jnp.float32),
        pltpu.SemaphoreType.DMA,
        pltpu.SemaphoreType.DMA,
        pltpu.SemaphoreType.DMA,
        pltpu.SemaphoreType.DMA,
        pltpu.SemaphoreType.DMA,
    ],
)
def _sc_edges(es_hbm, ed_hbm, src_hbm, dst_hbm, bl_hbm, a_hbm, b_hbm,
              out_hbm, es_v, ed_v,
              srcA, dstA, blA, aA, bA, vpA,
              srcB, dstB, blB, aB, bB, vpB,
              zero_v, acc_sp, semT, semA, semB, semSA, semSB):
    cid = lax.axis_index("c")
    sid = lax.axis_index("s")
    wid = sid * NC + cid
    hbm_ins = (src_hbm, dst_hbm, bl_hbm, a_hbm, b_hbm)
    setA = (srcA, dstA, blA, aA, bA)
    setB = (srcB, dstB, blB, aB, bB)

    def _start_in(ch, bufs, sem):
        for hbm, buf in zip(hbm_ins, bufs):
            pltpu.async_copy(hbm.at[wid, ch], buf, sem)

    def _wait_in(bufs, sem):
        for hbm, buf in zip(hbm_ins, bufs):
            pltpu.make_async_copy(hbm.at[wid, 0], buf, sem).wait()

    def _compute(src_v, dst_v, bl_v, a_v, b_v, vp_v):
        for r in range(ROWS):
            for g in range(LANES // 16):
                sl = pl.ds(g * 16, 16)
                s4 = src_v[r, sl] * 4
                d4 = dst_v[r, sl] * 4
                es0 = plsc.load_gather(es_v, [s4])
                es1 = plsc.load_gather(es_v, [s4 + 1])
                es2 = plsc.load_gather(es_v, [s4 + 2])
                es3 = plsc.load_gather(es_v, [s4 + 3])
                ed0 = plsc.load_gather(ed_v, [d4])
                ed1 = plsc.load_gather(ed_v, [d4 + 1])
                ed2 = plsc.load_gather(ed_v, [d4 + 2])
                ed3 = plsc.load_gather(ed_v, [d4 + 3])
                rr = bl_v[r, sl]
                rep = a_v[r, sl] * es0 * ed0 * jnp.exp(-es1 * ed1 * rr)
                att = b_v[r, sl] * es2 * ed2 * jnp.exp(-es3 * ed3 * rr)
                vp_v[r, sl] = rep - att

    tab0 = pltpu.async_copy(es_hbm, es_v, semT)
    tab1 = pltpu.async_copy(ed_hbm, ed_v, semT)
    _start_in(0, setA, semA)

    def _zero(i, carry):
        zero_v[pl.ds(i * 16, 16)] = jnp.zeros((16,), jnp.float32)
        return carry
    lax.fori_loop(0, N_SLICE // 16, _zero, 0)
    pltpu.sync_copy(zero_v, acc_sp.at[pl.ds(sid * N_SLICE, N_SLICE)])
    tab0.wait()
    tab1.wait()
    plsc.subcore_barrier()

    def _pair(i, carry):
        _start_in(2 * i + 1, setB, semB)
        _wait_in(setA, semA)
        _compute(*setA, vpA)
        scatA = [pltpu.async_copy(vpA.at[r], acc_sp.at[dstA.at[r]],
                                  semSA, add=True) for r in range(ROWS)]
        _wait_in(setB, semB)
        for c in scatA:
            c.wait()

        @pl.when(i < N_PAIR - 1)
        def _():
            _start_in(2 * i + 2, setA, semA)
        _compute(*setB, vpB)
        scatB = [pltpu.async_copy(vpB.at[r], acc_sp.at[dstB.at[r]],
                                  semSB, add=True) for r in range(ROWS)]
        for c in scatB:
            c.wait()
        return carry
    lax.fori_loop(0, N_PAIR, _pair, 0)

    plsc.subcore_barrier()
    pltpu.sync_copy(acc_sp.at[pl.ds(sid * N_SLICE, N_SLICE)],
                    out_hbm.at[cid, pl.ds(sid * N_SLICE, N_SLICE)])


def _tc_sum_body(x_ref, o_ref):
    o_ref[...] = x_ref[0, :] + x_ref[1, :]


_tc_sum = pl.pallas_call(
    _tc_sum_body,
    out_shape=jax.ShapeDtypeStruct((NPAD,), jnp.float32),
)


def kernel(node_features, bond_order, bondlength, edge_index, W_src, b_src, W_dst):
    e_extra = EPAD - N_EDGES
    bo_p = jnp.concatenate(
        [bond_order, jnp.zeros((e_extra,), jnp.float32)])
    bl_p = jnp.concatenate(
        [bondlength, jnp.full((e_extra,), 100.0, jnp.float32)])
    es, ed, a2, b2 = _tc_pre(
        node_features, W_src.T, b_src.reshape(1, 4), W_dst.T,
        bo_p.reshape(EPAD // 128, LANES), bl_p.reshape(EPAD // 128, LANES))

    es_f = jnp.pad(es, ((0, NPAD - N_NODES), (0, 0))).reshape(-1)
    ed_f = jnp.pad(ed, ((0, NPAD - N_NODES), (0, 0))).reshape(-1)
    zpad = jnp.zeros((e_extra,), jnp.int32)
    shp = (NW, N_CHUNK, ROWS, LANES)
    src_r = jnp.concatenate([edge_index[0], zpad]).reshape(shp)
    dst_r = jnp.concatenate([edge_index[1], zpad]).reshape(shp)
    bl_r = bl_p.reshape(shp)
    a_r = a2.reshape(shp)
    b_r = b2.reshape(shp)

    out2 = _sc_edges(es_f, ed_f, src_r, dst_r, bl_r, a_r, b_r)
    return _tc_sum(out2)[:N_NODES]
